# Initial kernel scaffold; baseline (speedup 1.0000x reference)
import jax
import jax.numpy as jnp
from jax.experimental import pallas as pl


def kernel(x, edge_index, sage_w_l, sage_b_l, sage_w_r, gat_w, gat_att_src, gat_att_dst, gat_bias):
    raise NotImplementedError("write your pallas kernel here")


# speedup vs baseline: 20.6442x; 20.6442x over previous
import functools

import jax
import jax.numpy as jnp
from jax import lax
from jax.experimental import pallas as pl
from jax.experimental.pallas import tpu as pltpu
from jax.experimental.pallas import tpu_sc as plsc

N = 10000
E = 320000
D = 128
DAUG = 144
W = 80
EPT = E // 16
B = 80
NBLK = EPT // B
TROWS = N // 16
ZROWS = 125


def _run_pass(tab_hbm, out_hbm, sidx, didx, rows, sems, acc, process,
              acc_lo, acc_w, out_lo, sid):
    def zrow(i, c):
        for kk in range(W // 16):
            rows[0][i, pl.ds(16 * kk, 16)] = jnp.zeros((16,), jnp.float32)
        return c

    lax.fori_loop(0, B, zrow, 0)
    for t in range(TROWS // B):
        pltpu.sync_copy(rows[0], acc.at[pl.ds(sid * TROWS + t * B, B)])
    rem = TROWS % B
    if rem:
        pltpu.sync_copy(
            rows[0].at[pl.ds(0, rem)],
            acc.at[pl.ds(sid * TROWS + (TROWS // B) * B, rem)])
    plsc.subcore_barrier()

    def g_start(j, b):
        pltpu.async_copy(tab_hbm.at[sidx.at[j]], rows[b], sems[b])

    def g_wait(j, b):
        pltpu.make_async_copy(tab_hbm.at[sidx.at[j]], rows[b], sems[b]).wait()

    def stage(j, b):
        g_wait(j, b)

        @pl.when(j + 1 < NBLK)
        def _():
            g_start(j + 1, 1 - b)

        process(j, rows[b])
        pltpu.sync_copy(rows[b], acc.at[didx.at[j]], add=True)

    g_start(0, 0)

    def body(jj, c):
        stage(2 * jj, 0)
        stage(2 * jj + 1, 1)
        return c

    lax.fori_loop(0, NBLK // 2, body, 0)
    plsc.subcore_barrier()
    pltpu.sync_copy(
        acc.at[pl.ds(sid * TROWS, TROWS), pl.ds(acc_lo, acc_w)],
        out_hbm.at[pl.ds(sid * TROWS, TROWS), pl.ds(out_lo, acc_w)],
    )


def _stage_idx(s_hbm, d_hbm, sidx, didx, sid):
    pltpu.sync_copy(s_hbm.at[pl.ds(sid * NBLK, NBLK)], sidx)
    pltpu.sync_copy(d_hbm.at[pl.ds(sid * NBLK, NBLK)], didx)


_SC_SCRATCH = [
    pltpu.VMEM((NBLK, B), jnp.int32),
    pltpu.VMEM((NBLK, B), jnp.int32),
    pltpu.VMEM((B, W), jnp.float32),
    pltpu.VMEM((B, W), jnp.float32),
    pltpu.VMEM_SHARED((N, W), jnp.float32),
    pltpu.SemaphoreType.DMA,
    pltpu.SemaphoreType.DMA,
]


def _sage_agg(t0, t1, src2d, dst2d):
    mesh = plsc.VectorSubcoreMesh(core_axis_name="c", subcore_axis_name="s")

    @functools.partial(
        pl.kernel,
        out_type=jax.ShapeDtypeStruct((N, DAUG), jnp.float32),
        mesh=mesh,
        compiler_params=pltpu.CompilerParams(use_tc_tiling_on_sc=False, needs_layout_passes=False),
        scratch_types=_SC_SCRATCH,
    )
    def k(t0_hbm, t1_hbm, s_hbm, d_hbm, out_hbm,
          sidx, didx, r0, r1, acc, m0, m1):
        cid = lax.axis_index("c")
        sid = lax.axis_index("s")
        _stage_idx(s_hbm, d_hbm, sidx, didx, sid)
        nop = lambda j, buf: None

        @pl.when(cid == 0)
        def _():
            _run_pass(t0_hbm, out_hbm, sidx, didx, (r0, r1), (m0, m1),
                      acc, nop, 0, W, 0, sid)

        @pl.when(cid == 1)
        def _():
            _run_pass(t1_hbm, out_hbm, sidx, didx, (r0, r1), (m0, m1),
                      acc, nop, 16, 64, 80, sid)

    return k(t0, t1, src2d, dst2d)


def _gat_agg(t0, t1, src2d, dst2d, a_s, a_d, mvec):
    mesh = plsc.VectorSubcoreMesh(core_axis_name="c", subcore_axis_name="s")

    @functools.partial(
        pl.kernel,
        out_type=jax.ShapeDtypeStruct((N, DAUG), jnp.float32),
        mesh=mesh,
        compiler_params=pltpu.CompilerParams(use_tc_tiling_on_sc=False, needs_layout_passes=False),
        scratch_types=_SC_SCRATCH + [
            pltpu.VMEM((N,), jnp.float32),
            pltpu.VMEM((N,), jnp.float32),
            pltpu.VMEM((B,), jnp.float32),
            pltpu.VMEM((16,), jnp.float32),
        ],
    )
    def k(t0_hbm, t1_hbm, s_hbm, d_hbm, as_hbm, ad_hbm, mv_hbm, out_hbm,
          sidx, didx, r0, r1, acc, m0, m1, astab, adtab, wbuf, mvv):
        cid = lax.axis_index("c")
        sid = lax.axis_index("s")
        _stage_idx(s_hbm, d_hbm, sidx, didx, sid)
        pltpu.sync_copy(as_hbm, astab)
        pltpu.sync_copy(ad_hbm, adtab)
        pltpu.sync_copy(mv_hbm, mvv)

        def process(j, buf):
            mv = mvv[...]

            def wblk(k2, c):
                si = sidx[j, pl.ds(16 * k2, 16)]
                di = didx[j, pl.ds(16 * k2, 16)]
                e = plsc.load_gather(astab, [si]) + plsc.load_gather(adtab, [di])
                e = jnp.where(e > 0.0, e, 0.2 * e) - mv
                wbuf[pl.ds(16 * k2, 16)] = jnp.exp(e)
                return c

            lax.fori_loop(0, B // 16, wblk, 0)

            def srow16(g, c):
                wv16 = wbuf[pl.ds(16 * g, 16)]
                for i2 in range(16):
                    wv = wv16[i2]
                    i = 16 * g + i2
                    for kk in range(W // 16):
                        buf[i, pl.ds(16 * kk, 16)] = (
                            buf[i, pl.ds(16 * kk, 16)] * wv)
                return c

            lax.fori_loop(0, B // 16, srow16, 0)

        @pl.when(cid == 0)
        def _():
            _run_pass(t0_hbm, out_hbm, sidx, didx, (r0, r1), (m0, m1),
                      acc, process, 0, W, 0, sid)

        @pl.when(cid == 1)
        def _():
            _run_pass(t1_hbm, out_hbm, sidx, didx, (r0, r1), (m0, m1),
                      acc, process, 16, 64, 80, sid)

    return k(t0, t1, src2d, dst2d, a_s, a_d, mvec)


def _tc_mid(agg, x, w_l, b_l, w_r, gat_w, att_s, att_d):
    def body(agg_ref, x_ref, wl_ref, bl_ref, wr_ref, gw_ref, as_ref, ad_ref,
             h_ref, asc_ref, adc_ref, m_ref):
        agg = agg_ref[...]
        col = lax.broadcasted_iota(jnp.int32, (N, DAUG), 1)
        cnt = jnp.sum(jnp.where(col == D, agg, 0.0), axis=1, keepdims=True)
        mean = agg[:, :D] / jnp.maximum(cnt, 1.0)
        dn = (((1,), (1,)), ((), ()))
        h1 = (lax.dot_general(mean, wl_ref[...], dn,
                              preferred_element_type=jnp.float32)
              + bl_ref[...]
              + lax.dot_general(x_ref[...], wr_ref[...], dn,
                                preferred_element_type=jnp.float32))
        h1 = jnp.maximum(h1, 0.0)
        h = lax.dot_general(h1, gw_ref[...], dn,
                            preferred_element_type=jnp.float32)
        a_s = jnp.sum(h * as_ref[...], axis=1, keepdims=True)
        a_d = jnp.sum(h * ad_ref[...], axis=1, keepdims=True)
        m = jnp.max(a_s) + jnp.max(a_d)
        m = jnp.where(m > 0.0, m, 0.2 * m)
        h_ref[...] = h
        asc_ref[...] = a_s
        adc_ref[...] = a_d
        m_ref[...] = jnp.full((1, 1), m, jnp.float32)

    return pl.pallas_call(
        body,
        out_shape=(
            jax.ShapeDtypeStruct((N, D), jnp.float32),
            jax.ShapeDtypeStruct((N, 1), jnp.float32),
            jax.ShapeDtypeStruct((N, 1), jnp.float32),
            jax.ShapeDtypeStruct((1, 1), jnp.float32),
        ),
    )(agg, x, w_l, b_l, w_r, gat_w, att_s, att_d)


def _tc_final(acc, h, a_s, a_d, m, bias):
    def body(acc_ref, h_ref, as_ref, ad_ref, m_ref, b_ref, out_ref):
        acc = acc_ref[...]
        col = lax.broadcasted_iota(jnp.int32, (N, DAUG), 1)
        s = jnp.sum(jnp.where(col == D, acc, 0.0), axis=1, keepdims=True)
        num = acc[:, :D]
        e = as_ref[...] + ad_ref[...]
        e = jnp.where(e > 0.0, e, 0.2 * e) - m_ref[0, 0]
        wself = jnp.exp(e)
        num = num + wself * h_ref[...]
        s = s + wself
        out = num / (s + 1e-16) + b_ref[...]
        mx = jnp.max(out, axis=1, keepdims=True)
        z = out - mx
        out_ref[...] = z - jnp.log(jnp.sum(jnp.exp(z), axis=1, keepdims=True))

    return pl.pallas_call(
        body,
        out_shape=jax.ShapeDtypeStruct((N, D), jnp.float32),
    )(acc, h, a_s, a_d, m, bias)


def kernel(x, edge_index, sage_w_l, sage_b_l, sage_w_r, gat_w, gat_att_src,
           gat_att_dst, gat_bias):
    ei = edge_index.astype(jnp.int32)
    src2d = ei[0].reshape(E // B, B)
    dst2d = ei[1].reshape(E // B, B)
    xaug = jnp.concatenate(
        [x, jnp.ones((N, 1), jnp.float32),
         jnp.zeros((N, DAUG - D - 1), jnp.float32)], axis=1)

    agg = _sage_agg(xaug[:, :W], xaug[:, DAUG - W:], src2d, dst2d)

    h, a_s, a_d, m = _tc_mid(
        agg, x, sage_w_l, sage_b_l.reshape(1, D), sage_w_r, gat_w,
        gat_att_src.reshape(1, D), gat_att_dst.reshape(1, D))

    haug = jnp.concatenate(
        [h, jnp.ones((N, 1), jnp.float32),
         jnp.zeros((N, DAUG - D - 1), jnp.float32)], axis=1)
    mvec = jnp.broadcast_to(m.reshape(()), (16,))

    acc = _gat_agg(haug[:, :W], haug[:, DAUG - W:], src2d, dst2d,
                   a_s.reshape(N), a_d.reshape(N), mvec)

    return _tc_final(acc, h, a_s, a_d, m, gat_bias.reshape(1, D))
